# Initial kernel scaffold; baseline (speedup 1.0000x reference)
import jax
import jax.numpy as jnp
from jax.experimental import pallas as pl


def kernel(boxes, classification):
    raise NotImplementedError("write your pallas kernel here")


# speedup vs baseline: 19.4229x; 19.4229x over previous
import jax
import jax.numpy as jnp
from jax.experimental import pallas as pl
from jax.experimental.pallas import tpu as pltpu

_SCORE_THRESHOLD = 0.05
_MAX_DETECTIONS = 300
_NMS_THRESHOLD = 0.5

_ROWS = 160
_LANES = 128
_P = _ROWS * _LANES


def _nms_body(boxes_ref, cls_ref, out_ref):
    X1 = boxes_ref[0, 0]
    Y1 = boxes_ref[0, 1]
    X2 = boxes_ref[0, 2]
    Y2 = boxes_ref[0, 3]
    AREA = (X2 - X1) * (Y2 - Y1)

    C = cls_ref.shape[1]
    best = cls_ref[0, 0]
    labf = jnp.zeros((_ROWS, _LANES), jnp.float32)
    for c in range(1, C):
        v = cls_ref[0, c]
        gt = v > best
        best = jnp.where(gt, v, best)
        labf = jnp.where(gt, jnp.float32(c), labf)

    NEG = jnp.float32(-jnp.inf)
    work0 = jnp.where(best > _SCORE_THRESHOLD, best, NEG)

    I = (jax.lax.broadcasted_iota(jnp.int32, (_ROWS, _LANES), 0) * _LANES
         + jax.lax.broadcasted_iota(jnp.int32, (_ROWS, _LANES), 1))
    lane = jax.lax.broadcasted_iota(jnp.int32, (1, _LANES), 1)

    def body(i, work):
        m = jnp.max(work)
        has = m > NEG
        idx = jnp.min(jnp.where(work == m, I, jnp.int32(1 << 30)))
        flag = I == idx

        def pick(a):
            return jnp.sum(jnp.where(flag, a, 0.0))

        x1b = pick(X1)
        y1b = pick(Y1)
        x2b = pick(X2)
        y2b = pick(Y2)
        lb = pick(labf)
        ab = (x2b - x1b) * (y2b - y1b)

        ix1 = jnp.maximum(X1, x1b)
        iy1 = jnp.maximum(Y1, y1b)
        ix2 = jnp.minimum(X2, x2b)
        iy2 = jnp.minimum(Y2, y2b)
        iw = jnp.maximum(ix2 - ix1, 0.0)
        ih = jnp.maximum(iy2 - iy1, 0.0)
        inter = iw * ih
        union = jnp.maximum(AREA + ab - inter, 1e-8)
        sup = (inter > _NMS_THRESHOLD * union) & has
        work = jnp.where(sup, NEG, work)

        row = jnp.full((1, _LANES), -1.0, jnp.float32)
        for j, v in enumerate((x1b, y1b, x2b, y2b, m, lb)):
            row = jnp.where(lane == j, v, row)
        row = jnp.where(has, row, jnp.float32(-1.0))
        out_ref[0, pl.ds(i, 1), :] = row
        return work

    jax.lax.fori_loop(0, _MAX_DETECTIONS, body, work0, unroll=2)


def kernel(boxes, classification):
    B, N, C = classification.shape
    bt = jnp.transpose(boxes, (0, 2, 1))
    bt = jnp.pad(bt, ((0, 0), (0, 0), (0, _P - N)))
    bt = bt.reshape(B, 4, _ROWS, _LANES)
    ct = jnp.transpose(classification, (0, 2, 1))
    ct = jnp.pad(ct, ((0, 0), (0, 0), (0, _P - N)), constant_values=-1.0)
    ct = ct.reshape(B, C, _ROWS, _LANES)

    out = pl.pallas_call(
        _nms_body,
        grid=(B,),
        in_specs=[
            pl.BlockSpec((1, 4, _ROWS, _LANES), lambda b: (b, 0, 0, 0)),
            pl.BlockSpec((1, C, _ROWS, _LANES), lambda b: (b, 0, 0, 0)),
        ],
        out_specs=pl.BlockSpec((1, 304, _LANES), lambda b: (b, 0, 0)),
        out_shape=jax.ShapeDtypeStruct((B, 304, _LANES), jnp.float32),
        compiler_params=pltpu.CompilerParams(
            dimension_semantics=("arbitrary",),
        ),
    )(bt, ct)

    out_boxes = out[:, :_MAX_DETECTIONS, 0:4]
    out_scores = out[:, :_MAX_DETECTIONS, 4]
    out_labels = out[:, :_MAX_DETECTIONS, 5].astype(jnp.int32)
    return out_boxes, out_scores, out_labels
